# baseline (speedup 1.0000x reference)
import jax
import jax.numpy as jnp
from jax.experimental import pallas as pl
from jax.experimental.pallas import tpu as pltpu

_REINIT_P = 0.05
_N, _H, _W = 10000, 250, 100
_B = 128
_R = 80
_NR, _NC = 80, 125


_K = 8
_RC = 25
_NG = _N // (_K * _RC)


def _copy_body(buf_hbm, newbuf_hbm, slots, in_sems, out_sems):
    def in_copy(c, k):
        sl = pl.ds(c * _RC, _RC)
        return pltpu.make_async_copy(buf_hbm.at[sl], slots.at[k], in_sems.at[k])

    def out_copy(c, k):
        sl = pl.ds(c * _RC, _RC)
        return pltpu.make_async_copy(slots.at[k], newbuf_hbm.at[sl], out_sems.at[k])

    full = pltpu.make_async_copy(buf_hbm, newbuf_hbm, in_sems.at[0])
    full.start()
    full.wait()
    del out_copy, in_copy, slots, out_sems


def _gs_body(idx_ref, u_ref, buf_row, noise_row, newbuf_hbm,
             out_row, newbuf_row):
    del newbuf_hbm
    b = pl.program_id(0)
    reinit = u_ref[b] < _REINIT_P

    @pl.when(reinit)
    def _():
        out_row[...] = noise_row[...]
        newbuf_row[...] = noise_row[...]

    @pl.when(jnp.logical_not(reinit))
    def _():
        out_row[...] = buf_row[...]
        newbuf_row[...] = buf_row[...]


def _ns_body(idx_ref, u_ref, ns_ref, outns_ref, newns_ref):
    ns = ns_ref[...]
    rowid = jax.lax.broadcasted_iota(jnp.int32, (_NR, _NC), 0) * _NC + \
        jax.lax.broadcasted_iota(jnp.int32, (_NR, _NC), 1)
    bid = jax.lax.broadcasted_iota(jnp.int32, (1, _B), 1)

    def body(b, carry):
        acc, newns = carry
        i = idx_ref[b]
        reinit = u_ref[b] < _REINIT_P
        m = rowid == i
        val = jnp.where(reinit, 0.0, jnp.sum(jnp.where(m, ns, 0.0)))
        acc = jnp.where(bid == b, val, acc)
        newns = jnp.where(m, val, newns)
        return acc, newns

    acc, newns = jax.lax.fori_loop(
        0, _B, body, (jnp.zeros((1, _B), jnp.float32), ns))
    outns_ref[...] = acc
    newns_ref[...] = newns


def kernel(buffer, buffer_numsteps, noise, u, idx):
    idx = idx.astype(jnp.int32)

    newbuf0 = pl.pallas_call(
        _copy_body,
        out_shape=jax.ShapeDtypeStruct((_N, _H, _W), jnp.float32),
        in_specs=[pl.BlockSpec(memory_space=pl.ANY)],
        out_specs=pl.BlockSpec(memory_space=pl.ANY),
        scratch_shapes=[
            pltpu.VMEM((_K, _RC, _H, _W), jnp.float32),
            pltpu.SemaphoreType.DMA((_K,)),
            pltpu.SemaphoreType.DMA((_K,)),
        ],
    )(buffer)

    grid_spec = pltpu.PrefetchScalarGridSpec(
        num_scalar_prefetch=2,
        grid=(_B,),
        in_specs=[
            pl.BlockSpec((1, _H, _W), lambda b, idx_r, u_r: (idx_r[b], 0, 0)),
            pl.BlockSpec((1, _H, _W), lambda b, idx_r, u_r: (b, 0, 0)),
            pl.BlockSpec(memory_space=pl.ANY),
        ],
        out_specs=[
            pl.BlockSpec((1, _H, _W), lambda b, idx_r, u_r: (b, 0, 0)),
            pl.BlockSpec((1, _H, _W), lambda b, idx_r, u_r: (idx_r[b], 0, 0)),
        ],
    )
    out, newbuf = pl.pallas_call(
        _gs_body,
        grid_spec=grid_spec,
        out_shape=[
            jax.ShapeDtypeStruct((_B, _H, _W), jnp.float32),
            jax.ShapeDtypeStruct((_N, _H, _W), jnp.float32),
        ],
        input_output_aliases={4: 1},
    )(idx, u, buffer, noise, newbuf0)

    outns, newns = pl.pallas_call(
        _ns_body,
        grid_spec=pltpu.PrefetchScalarGridSpec(
            num_scalar_prefetch=2,
            grid=(1,),
            in_specs=[pl.BlockSpec((_NR, _NC), lambda i, idx_r, u_r: (0, 0))],
            out_specs=[
                pl.BlockSpec((1, _B), lambda i, idx_r, u_r: (0, 0)),
                pl.BlockSpec((_NR, _NC), lambda i, idx_r, u_r: (0, 0)),
            ],
        ),
        out_shape=[
            jax.ShapeDtypeStruct((1, _B), jnp.float32),
            jax.ShapeDtypeStruct((_NR, _NC), jnp.float32),
        ],
    )(idx, u, buffer_numsteps.reshape(_NR, _NC))

    return (out, outns.reshape(_B), newbuf, newns.reshape(_N))


# speedup vs baseline: 13.2657x; 13.2657x over previous
import jax
import jax.numpy as jnp
from jax.experimental import pallas as pl
from jax.experimental.pallas import tpu as pltpu

_REINIT_P = 0.05
_N, _H, _W = 10000, 250, 100
_B = 128
_R = 80
_NR, _NC = 80, 125


_K = 16
_D = 8
_RC = 16
_NCH = _N // _RC


def _copy_body(buf_hbm, newbuf_hbm, slots, in_sems, out_sems):
    def in_copy(c, k):
        sl = pl.ds(c * _RC, _RC)
        return pltpu.make_async_copy(buf_hbm.at[sl], slots.at[k], in_sems.at[k])

    def out_copy(c, k):
        sl = pl.ds(c * _RC, _RC)
        return pltpu.make_async_copy(slots.at[k], newbuf_hbm.at[sl], out_sems.at[k])

    def step(c, _):
        @pl.when(c < _NCH)
        def _():
            s = jax.lax.rem(c, _K)

            @pl.when(c >= _K)
            def _():
                out_copy(c - _K, s).wait()

            in_copy(c, s).start()

        d = c - _D

        @pl.when(jnp.logical_and(d >= 0, d < _NCH))
        def _():
            s2 = jax.lax.rem(d, _K)
            in_copy(d, s2).wait()
            out_copy(d, s2).start()

        return 0

    jax.lax.fori_loop(0, _NCH + _D, step, 0)
    for k in range(_K):
        c = _NCH - _K + k
        out_copy(c, c % _K).wait()


def _gs_body(idx_ref, u_ref, buf_row, noise_row, newbuf_hbm,
             out_row, newbuf_row):
    del newbuf_hbm
    b = pl.program_id(0)
    reinit = u_ref[b] < _REINIT_P

    @pl.when(reinit)
    def _():
        out_row[...] = noise_row[...]
        newbuf_row[...] = noise_row[...]

    @pl.when(jnp.logical_not(reinit))
    def _():
        out_row[...] = buf_row[...]
        newbuf_row[...] = buf_row[...]


def _ns_body(idx_ref, u_ref, ns_ref, outns_ref, newns_ref):
    ns = ns_ref[...]
    rowid = jax.lax.broadcasted_iota(jnp.int32, (_NR, _NC), 0) * _NC + \
        jax.lax.broadcasted_iota(jnp.int32, (_NR, _NC), 1)
    bid = jax.lax.broadcasted_iota(jnp.int32, (1, _B), 1)

    def body(b, carry):
        acc, newns = carry
        i = idx_ref[b]
        reinit = u_ref[b] < _REINIT_P
        m = rowid == i
        val = jnp.where(reinit, 0.0, jnp.sum(jnp.where(m, ns, 0.0)))
        acc = jnp.where(bid == b, val, acc)
        newns = jnp.where(m, val, newns)
        return acc, newns

    acc, newns = jax.lax.fori_loop(
        0, _B, body, (jnp.zeros((1, _B), jnp.float32), ns))
    outns_ref[...] = acc
    newns_ref[...] = newns


def kernel(buffer, buffer_numsteps, noise, u, idx):
    idx = idx.astype(jnp.int32)

    newbuf0 = pl.pallas_call(
        _copy_body,
        out_shape=jax.ShapeDtypeStruct((_N, _H, _W), jnp.float32),
        in_specs=[pl.BlockSpec(memory_space=pl.ANY)],
        out_specs=pl.BlockSpec(memory_space=pl.ANY),
        scratch_shapes=[
            pltpu.VMEM((_K, _RC, _H, _W), jnp.float32),
            pltpu.SemaphoreType.DMA((_K,)),
            pltpu.SemaphoreType.DMA((_K,)),
        ],
        compiler_params=pltpu.CompilerParams(vmem_limit_bytes=128 * 1024 * 1024),
    )(buffer)

    grid_spec = pltpu.PrefetchScalarGridSpec(
        num_scalar_prefetch=2,
        grid=(_B,),
        in_specs=[
            pl.BlockSpec((1, _H, _W), lambda b, idx_r, u_r: (idx_r[b], 0, 0)),
            pl.BlockSpec((1, _H, _W), lambda b, idx_r, u_r: (b, 0, 0)),
            pl.BlockSpec(memory_space=pl.ANY),
        ],
        out_specs=[
            pl.BlockSpec((1, _H, _W), lambda b, idx_r, u_r: (b, 0, 0)),
            pl.BlockSpec((1, _H, _W), lambda b, idx_r, u_r: (idx_r[b], 0, 0)),
        ],
    )
    out, newbuf = pl.pallas_call(
        _gs_body,
        grid_spec=grid_spec,
        out_shape=[
            jax.ShapeDtypeStruct((_B, _H, _W), jnp.float32),
            jax.ShapeDtypeStruct((_N, _H, _W), jnp.float32),
        ],
        input_output_aliases={4: 1},
    )(idx, u, buffer, noise, newbuf0)

    outns, newns = pl.pallas_call(
        _ns_body,
        grid_spec=pltpu.PrefetchScalarGridSpec(
            num_scalar_prefetch=2,
            grid=(1,),
            in_specs=[pl.BlockSpec((_NR, _NC), lambda i, idx_r, u_r: (0, 0))],
            out_specs=[
                pl.BlockSpec((1, _B), lambda i, idx_r, u_r: (0, 0)),
                pl.BlockSpec((_NR, _NC), lambda i, idx_r, u_r: (0, 0)),
            ],
        ),
        out_shape=[
            jax.ShapeDtypeStruct((1, _B), jnp.float32),
            jax.ShapeDtypeStruct((_NR, _NC), jnp.float32),
        ],
    )(idx, u, buffer_numsteps.reshape(_NR, _NC))

    return (out, outns.reshape(_B), newbuf, newns.reshape(_N))


# speedup vs baseline: 17.6775x; 1.3326x over previous
import jax
import jax.numpy as jnp
from jax.experimental import pallas as pl
from jax.experimental.pallas import tpu as pltpu

_REINIT_P = 0.05
_N, _H, _W = 10000, 250, 100
_B = 128
_NR, _NC = 80, 125


def _gather_body(idx_ref, u_ref, buf_row, noise_row, out_row):
    b = pl.program_id(0)
    reinit = u_ref[b] < _REINIT_P

    @pl.when(reinit)
    def _():
        out_row[...] = noise_row[...]

    @pl.when(jnp.logical_not(reinit))
    def _():
        out_row[...] = buf_row[...]


def _ns_body(idx_ref, u_ref, ns_ref, idxv_ref, uv_ref,
             outns_ref, newns_ref, wmask_ref):
    ns = ns_ref[...]
    rowid = jax.lax.broadcasted_iota(jnp.int32, (_NR, _NC), 0) * _NC + \
        jax.lax.broadcasted_iota(jnp.int32, (_NR, _NC), 1)
    bid = jax.lax.broadcasted_iota(jnp.int32, (1, _B), 1)

    def body(b, carry):
        acc, newns = carry
        i = idx_ref[b]
        reinit = u_ref[b] < _REINIT_P
        m = rowid == i
        val = jnp.where(reinit, 0.0, jnp.sum(jnp.where(m, ns, 0.0)))
        acc = jnp.where(bid == b, val, acc)
        newns = jnp.where(m, val, newns)
        return acc, newns

    acc, newns = jax.lax.fori_loop(
        0, _B, body, (jnp.zeros((1, _B), jnp.float32), ns))
    outns_ref[...] = acc
    newns_ref[...] = newns

    idxv = idxv_ref[...]
    eq = idxv.reshape(_B, 1) == idxv
    later = bid > bid.reshape(_B, 1)
    has_later = jnp.sum(
        jnp.where(jnp.logical_and(eq, later), 1, 0), axis=1, keepdims=True)
    reinit_v = uv_ref[...] < _REINIT_P
    win = jnp.logical_and(has_later.reshape(1, _B) == 0, reinit_v)
    wmask_ref[...] = win.astype(jnp.int32)


def _scatter_body(idx_ref, w_ref, noise_hbm, buf_alias, newbuf_hbm, sem):
    del buf_alias

    def row_copy(b):
        return pltpu.make_async_copy(
            noise_hbm.at[pl.ds(b, 1)], newbuf_hbm.at[pl.ds(idx_ref[b], 1)], sem)

    def start(b, _):
        @pl.when(w_ref[b] == 1)
        def _():
            row_copy(b).start()
        return 0

    def wait(b, _):
        @pl.when(w_ref[b] == 1)
        def _():
            row_copy(b).wait()
        return 0

    jax.lax.fori_loop(0, _B, start, 0)
    jax.lax.fori_loop(0, _B, wait, 0)


def kernel(buffer, buffer_numsteps, noise, u, idx):
    idx = idx.astype(jnp.int32)

    outns, newns, wmask = pl.pallas_call(
        _ns_body,
        grid_spec=pltpu.PrefetchScalarGridSpec(
            num_scalar_prefetch=2,
            grid=(1,),
            in_specs=[
                pl.BlockSpec((_NR, _NC), lambda i, idx_r, u_r: (0, 0)),
                pl.BlockSpec((1, _B), lambda i, idx_r, u_r: (0, 0)),
                pl.BlockSpec((1, _B), lambda i, idx_r, u_r: (0, 0)),
            ],
            out_specs=[
                pl.BlockSpec((1, _B), lambda i, idx_r, u_r: (0, 0)),
                pl.BlockSpec((_NR, _NC), lambda i, idx_r, u_r: (0, 0)),
                pl.BlockSpec((1, _B), lambda i, idx_r, u_r: (0, 0)),
            ],
        ),
        out_shape=[
            jax.ShapeDtypeStruct((1, _B), jnp.float32),
            jax.ShapeDtypeStruct((_NR, _NC), jnp.float32),
            jax.ShapeDtypeStruct((1, _B), jnp.int32),
        ],
    )(idx, u, buffer_numsteps.reshape(_NR, _NC), idx.reshape(1, _B),
      u.reshape(1, _B))

    out = pl.pallas_call(
        _gather_body,
        grid_spec=pltpu.PrefetchScalarGridSpec(
            num_scalar_prefetch=2,
            grid=(_B,),
            in_specs=[
                pl.BlockSpec((1, _H, _W), lambda b, idx_r, u_r: (idx_r[b], 0, 0)),
                pl.BlockSpec((1, _H, _W), lambda b, idx_r, u_r: (b, 0, 0)),
            ],
            out_specs=pl.BlockSpec((1, _H, _W), lambda b, idx_r, u_r: (b, 0, 0)),
        ),
        out_shape=jax.ShapeDtypeStruct((_B, _H, _W), jnp.float32),
    )(idx, u, buffer, noise)

    newbuf = pl.pallas_call(
        _scatter_body,
        grid_spec=pltpu.PrefetchScalarGridSpec(
            num_scalar_prefetch=2,
            grid=(1,),
            in_specs=[
                pl.BlockSpec(memory_space=pl.ANY),
                pl.BlockSpec(memory_space=pl.ANY),
            ],
            out_specs=pl.BlockSpec(memory_space=pl.ANY),
            scratch_shapes=[pltpu.SemaphoreType.DMA],
        ),
        out_shape=jax.ShapeDtypeStruct((_N, _H, _W), jnp.float32),
        input_output_aliases={3: 0},
    )(idx, wmask.reshape(_B), noise, buffer)

    return (out, outns.reshape(_B), newbuf, newns.reshape(_N))


# speedup vs baseline: 18.3784x; 1.0396x over previous
import jax
import jax.numpy as jnp
from jax.experimental import pallas as pl
from jax.experimental.pallas import tpu as pltpu

_REINIT_P = 0.05
_N, _H, _W = 10000, 250, 100
_B = 128
_NR, _NC = 80, 125


def _gather_body(idx_ref, u_ref, buf_hbm, noise_hbm, out_hbm, slots, sems):
    def noise_copy(b):
        return pltpu.make_async_copy(
            noise_hbm.at[pl.ds(b, 1)], slots.at[pl.ds(b, 1)], sems.at[0])

    def buf_copy(b):
        return pltpu.make_async_copy(
            buf_hbm.at[pl.ds(idx_ref[b], 1)], slots.at[pl.ds(b, 1)], sems.at[0])

    def start(b, _):
        reinit = u_ref[b] < _REINIT_P

        @pl.when(reinit)
        def _():
            noise_copy(b).start()

        @pl.when(jnp.logical_not(reinit))
        def _():
            buf_copy(b).start()

        return 0

    def wait(b, _):
        reinit = u_ref[b] < _REINIT_P

        @pl.when(reinit)
        def _():
            noise_copy(b).wait()

        @pl.when(jnp.logical_not(reinit))
        def _():
            buf_copy(b).wait()

        return 0

    jax.lax.fori_loop(0, _B, start, 0)
    jax.lax.fori_loop(0, _B, wait, 0)
    big = pltpu.make_async_copy(slots, out_hbm, sems.at[1])
    big.start()
    big.wait()


def _ns_body(idx_ref, u_ref, nssm_ref, ns_ref, idxv_ref, uv_ref,
             outns_ref, newns_ref, wmask_ref):
    rowid = jax.lax.broadcasted_iota(jnp.int32, (_NR, _NC), 0) * _NC + \
        jax.lax.broadcasted_iota(jnp.int32, (_NR, _NC), 1)
    bid = jax.lax.broadcasted_iota(jnp.int32, (1, _B), 1)

    def body(b, carry):
        acc, newns = carry
        i = idx_ref[b]
        reinit = u_ref[b] < _REINIT_P
        val = jnp.where(reinit, 0.0, nssm_ref[i])
        acc = jnp.where(bid == b, val, acc)
        newns = jnp.where(jnp.logical_and(reinit, rowid == i), 0.0, newns)
        return acc, newns

    acc, newns = jax.lax.fori_loop(
        0, _B, body, (jnp.zeros((1, _B), jnp.float32), ns_ref[...]))
    outns_ref[...] = acc
    newns_ref[...] = newns

    idxv = idxv_ref[...]
    eq = idxv.reshape(_B, 1) == idxv
    later = bid > bid.reshape(_B, 1)
    has_later = jnp.sum(
        jnp.where(jnp.logical_and(eq, later), 1, 0), axis=1, keepdims=True)
    reinit_v = uv_ref[...] < _REINIT_P
    win = jnp.logical_and(has_later.reshape(1, _B) == 0, reinit_v)
    wmask_ref[...] = win.astype(jnp.int32)


def _scatter_body(idx_ref, w_ref, noise_hbm, buf_alias, newbuf_hbm, sem):
    del buf_alias

    def row_copy(b):
        return pltpu.make_async_copy(
            noise_hbm.at[pl.ds(b, 1)], newbuf_hbm.at[pl.ds(idx_ref[b], 1)], sem)

    def start(b, _):
        @pl.when(w_ref[b] == 1)
        def _():
            row_copy(b).start()
        return 0

    def wait(b, _):
        @pl.when(w_ref[b] == 1)
        def _():
            row_copy(b).wait()
        return 0

    jax.lax.fori_loop(0, _B, start, 0)
    jax.lax.fori_loop(0, _B, wait, 0)


def kernel(buffer, buffer_numsteps, noise, u, idx):
    idx = idx.astype(jnp.int32)

    outns, newns, wmask = pl.pallas_call(
        _ns_body,
        grid_spec=pltpu.PrefetchScalarGridSpec(
            num_scalar_prefetch=3,
            grid=(1,),
            in_specs=[
                pl.BlockSpec((_NR, _NC), lambda i, *_: (0, 0)),
                pl.BlockSpec((1, _B), lambda i, *_: (0, 0)),
                pl.BlockSpec((1, _B), lambda i, *_: (0, 0)),
            ],
            out_specs=[
                pl.BlockSpec((1, _B), lambda i, *_: (0, 0)),
                pl.BlockSpec((_NR, _NC), lambda i, *_: (0, 0)),
                pl.BlockSpec((1, _B), lambda i, *_: (0, 0)),
            ],
        ),
        out_shape=[
            jax.ShapeDtypeStruct((1, _B), jnp.float32),
            jax.ShapeDtypeStruct((_NR, _NC), jnp.float32),
            jax.ShapeDtypeStruct((1, _B), jnp.int32),
        ],
    )(idx, u, buffer_numsteps, buffer_numsteps.reshape(_NR, _NC),
      idx.reshape(1, _B), u.reshape(1, _B))

    out = pl.pallas_call(
        _gather_body,
        grid_spec=pltpu.PrefetchScalarGridSpec(
            num_scalar_prefetch=2,
            grid=(1,),
            in_specs=[
                pl.BlockSpec(memory_space=pl.ANY),
                pl.BlockSpec(memory_space=pl.ANY),
            ],
            out_specs=pl.BlockSpec(memory_space=pl.ANY),
            scratch_shapes=[
                pltpu.VMEM((_B, _H, _W), jnp.float32),
                pltpu.SemaphoreType.DMA((2,)),
            ],
        ),
        out_shape=jax.ShapeDtypeStruct((_B, _H, _W), jnp.float32),
        compiler_params=pltpu.CompilerParams(
            vmem_limit_bytes=64 * 1024 * 1024),
    )(idx, u, buffer, noise)

    newbuf = pl.pallas_call(
        _scatter_body,
        grid_spec=pltpu.PrefetchScalarGridSpec(
            num_scalar_prefetch=2,
            grid=(1,),
            in_specs=[
                pl.BlockSpec(memory_space=pl.ANY),
                pl.BlockSpec(memory_space=pl.ANY),
            ],
            out_specs=pl.BlockSpec(memory_space=pl.ANY),
            scratch_shapes=[pltpu.SemaphoreType.DMA],
        ),
        out_shape=jax.ShapeDtypeStruct((_N, _H, _W), jnp.float32),
        input_output_aliases={3: 0},
    )(idx, wmask.reshape(_B), noise, buffer)

    return (out, outns.reshape(_B), newbuf, newns.reshape(_N))
